# baseline (speedup 1.0000x reference)
import functools
import jax
import jax.numpy as jnp
from jax import lax
from jax.experimental import pallas as pl
from jax.experimental.pallas import tpu as pltpu
from jax.experimental.pallas import tpu_sc as plsc

N = 10000
NP = 10240
D = 128
NOUT = 10
ROWB = 256
GRID = NP // ROWB

NEG = -1e30
E_TOT = 330000

NTILES = 32
EB = 128
EP = 335872
NBLK = EP // EB
WT = 72


def _tcA_body(x_ref, w_ref, out_ref, gm_ref):
    i = pl.program_id(0)
    r = jnp.dot(x_ref[...], w_ref[...], preferred_element_type=jnp.float32)
    out_ref[...] = r
    cm = jnp.max(r, axis=0, keepdims=True)
    @pl.when(i == 0)
    def _():
        gm_ref[...] = jnp.full_like(gm_ref, NEG)
    gm_ref[0:1, :] = jnp.maximum(gm_ref[0:1, :], cm)


def _tcA(xpad, wcat):
    return pl.pallas_call(
        _tcA_body,
        grid=(GRID,),
        in_specs=[
            pl.BlockSpec((ROWB, 128), lambda i: (i, 0)),
            pl.BlockSpec((128, 128), lambda i: (0, 0)),
        ],
        out_specs=[
            pl.BlockSpec((ROWB, 128), lambda i: (i, 0)),
            pl.BlockSpec((8, 128), lambda i: (0, 0)),
        ],
        out_shape=[
            jax.ShapeDtypeStruct((NP, 128), jnp.float32),
            jax.ShapeDtypeStruct((8, 128), jnp.float32),
        ],
        compiler_params=pltpu.CompilerParams(
            dimension_semantics=("arbitrary",)),
    )(xpad, wcat)


def _tcB_body(p_ref, b_ref, rep_ref, w2_ref, out_ref, gm_ref):
    i = pl.program_id(0)
    r1 = p_ref[0] + p_ref[1]
    acc = r1[:, :64]
    den = r1[:, 64:72]
    drep = jnp.dot(den, rep_ref[...], preferred_element_type=jnp.float32)
    h1 = jnp.maximum(acc / (drep + 1e-16) + b_ref[0:1, :], 0.0)
    r = jnp.dot(h1, w2_ref[...], preferred_element_type=jnp.float32)
    out_ref[...] = r
    cm = jnp.max(r, axis=0, keepdims=True)
    @pl.when(i == 0)
    def _():
        gm_ref[...] = jnp.full_like(gm_ref, NEG)
    gm_ref[0:1, :] = jnp.maximum(gm_ref[0:1, :], cm)


def _tcB(p, b1, rep8, w2cat):
    return pl.pallas_call(
        _tcB_body,
        grid=(GRID,),
        in_specs=[
            pl.BlockSpec((2, ROWB, 72), lambda i: (0, i, 0)),
            pl.BlockSpec((1, 64), lambda i: (0, 0)),
            pl.BlockSpec((8, 64), lambda i: (0, 0)),
            pl.BlockSpec((64, 128), lambda i: (0, 0)),
        ],
        out_specs=[
            pl.BlockSpec((ROWB, 128), lambda i: (i, 0)),
            pl.BlockSpec((8, 128), lambda i: (0, 0)),
        ],
        out_shape=[
            jax.ShapeDtypeStruct((NP, 128), jnp.float32),
            jax.ShapeDtypeStruct((8, 128), jnp.float32),
        ],
        compiler_params=pltpu.CompilerParams(
            dimension_semantics=("arbitrary",)),
    )(p, b1, rep8, w2cat)


def _tcC_body(p_ref, b_ref, pick_ref, wl_ref, bl_ref, out_ref):
    r2 = p_ref[0] + p_ref[1]
    acc = r2[:, :64]
    den8 = r2[:, 64:72]
    d = jnp.dot(den8, pick_ref[...], preferred_element_type=jnp.float32)
    out2 = jnp.maximum(acc / (d + 1e-16) + b_ref[0:1, :], 0.0)
    logits = jnp.dot(out2, wl_ref[...], preferred_element_type=jnp.float32)
    logits = logits + bl_ref[0:1, :]
    m = jnp.max(logits, axis=1, keepdims=True)
    s = logits - m
    lse = jnp.log(jnp.sum(jnp.exp(s), axis=1, keepdims=True))
    out_ref[...] = s - lse


def _tcC(p, b2, pick8, wlpad, blpad):
    return pl.pallas_call(
        _tcC_body,
        grid=(GRID,),
        in_specs=[
            pl.BlockSpec((2, ROWB, 72), lambda i: (0, i, 0)),
            pl.BlockSpec((1, 64), lambda i: (0, 0)),
            pl.BlockSpec((8, 64), lambda i: (0, 0)),
            pl.BlockSpec((64, 128), lambda i: (0, 0)),
            pl.BlockSpec((1, 128), lambda i: (0, 0)),
        ],
        out_specs=pl.BlockSpec((ROWB, 128), lambda i: (i, 0)),
        out_shape=jax.ShapeDtypeStruct((NP, 128), jnp.float32),
        compiler_params=pltpu.CompilerParams(
            dimension_semantics=("arbitrary",)),
    )(p, b2, pick8, wlpad, blpad)


def _make_sc_edge(H):
    mesh = plsc.VectorSubcoreMesh(core_axis_name="c", subcore_axis_name="s")
    nper = NBLK // NTILES
    half = nper // 2
    rows_per_sub = NP // 16

    buf_t = [pltpu.VMEM((EB, WT), jnp.float32)] * 2
    buf_a = [pltpu.VMEM((EB, 8), jnp.float32)] * 2
    buf_si = [pltpu.VMEM((EB,), jnp.int32)] * 2
    buf_di = [pltpu.VMEM((EB,), jnp.int32)] * 2

    @functools.partial(
        pl.kernel,
        out_type=jax.ShapeDtypeStruct((2, NP, WT), jnp.float32),
        mesh=mesh,
        scratch_types=buf_si + buf_di + buf_t + buf_a + [
            pltpu.VMEM((EB, 8), jnp.float32),
            pltpu.VMEM((16,), jnp.float32),
            pltpu.VMEM_SHARED((NP, WT), jnp.float32),
            pltpu.SemaphoreType.DMA,
            pltpu.SemaphoreType.DMA,
            pltpu.SemaphoreType.DMA,
            pltpu.SemaphoreType.DMA,
            pltpu.SemaphoreType.DMA,
            pltpu.SemaphoreType.DMA,
            pltpu.SemaphoreType.DMA,
        ],
        compiler_params=pltpu.CompilerParams(
            needs_layout_passes=False, use_tc_tiling_on_sc=False),
    )
    def sc_edge(t_hbm, a_hbm, src_hbm, dst_hbm, g_hbm, out_hbm,
                si0, si1, di0, di1, t0, t1, a0, a1,
                w_v, g_v, acc, semi, semt0, semt1, sema0, sema1,
                sems0, sems1):
        c = lax.axis_index("c")
        s = lax.axis_index("s")
        wid = s * 2 + c
        ii = lax.iota(jnp.int32, 16)
        zero16 = jnp.zeros((16,), jnp.float32)
        si = [si0, si1]
        di = [di0, di1]
        tb = [t0, t1]
        ab = [a0, a1]
        semt = [semt0, semt1]
        sema = [sema0, sema1]
        semsc = [sems0, sems1]

        pltpu.sync_copy(g_hbm, g_v)
        gvec = g_v[...]

        def _zrow(e, _):
            for q in range(4):
                t0[e, pl.ds(16 * q, 16)] = zero16
            return 0
        lax.fori_loop(0, EB, _zrow, 0)

        def _ztail(i, _):
            rows = 2 * i + jnp.where(ii >= 8, 1, 0)
            plsc.store_scatter(t0, [rows, 64 + (ii & 7)], zero16)
            return 0
        lax.fori_loop(0, EB // 2, _ztail, 0)

        rowbase = s * rows_per_sub
        for k in range(rows_per_sub // EB):
            pltpu.sync_copy(t0, acc.at[pl.ds(rowbase + k * EB, EB)])
        plsc.subcore_barrier()

        pair_rows = jnp.where(ii >= 8, 1, 0)
        head_cols = ii & 7
        col64 = jnp.full((16,), 64, jnp.int32)
        col0 = jnp.zeros((16,), jnp.int32)

        def issue(b, k):
            base = (k * NTILES + wid) * EB
            c1 = pltpu.async_copy(src_hbm.at[pl.ds(base, EB)], si[b], semi)
            c2 = pltpu.async_copy(dst_hbm.at[pl.ds(base, EB)], di[b], semi)
            c1.wait()
            c2.wait()
            pltpu.async_copy(t_hbm.at[si[b]], tb[b], semt[b])
            pltpu.async_copy(a_hbm.at[di[b]], ab[b], sema[b])

        def wait_gathers(b):
            pltpu.make_async_copy(t_hbm.at[si[b]], tb[b], semt[b]).wait()
            pltpu.make_async_copy(a_hbm.at[di[b]], ab[b], sema[b]).wait()

        def compute_scatter(b):
            t_v = tb[b]
            a_v = ab[b]
            if H == 8:
                @plsc.parallel_loop(0, EB // 2, unroll=4)
                def wbody(i):
                    rows = 2 * i + pair_rows
                    asv = plsc.load_gather(t_v, [rows, 64 + head_cols])
                    adv = plsc.load_gather(a_v, [rows, head_cols])
                    t = asv + adv
                    t = jnp.maximum(t, 0.2 * t)
                    w = jnp.exp(t - gvec)
                    plsc.store_scatter(w_v, [rows, head_cols], w)
                    plsc.store_scatter(t_v, [rows, 64 + head_cols], w)

                @plsc.parallel_loop(0, EB, unroll=8)
                def mbody(e):
                    erows = jnp.full((16,), e, jnp.int32)
                    for q in range(4):
                        wexp = plsc.load_gather(
                            w_v, [erows, 2 * q + pair_rows])
                        t_v[e, pl.ds(16 * q, 16)] = (
                            t_v[e, pl.ds(16 * q, 16)] * wexp)
            else:
                @plsc.parallel_loop(0, EB // 16, unroll=2)
                def wbody(i):
                    rows = 16 * i + ii
                    asv = plsc.load_gather(t_v, [rows, col64])
                    adv = plsc.load_gather(a_v, [rows, col0])
                    t = asv + adv
                    t = jnp.maximum(t, 0.2 * t)
                    w = jnp.exp(t - gvec)
                    plsc.store_scatter(w_v, [rows, col0], w)
                    plsc.store_scatter(t_v, [rows, col64], w)

                @plsc.parallel_loop(0, EB, unroll=8)
                def mbody(e):
                    erows = jnp.full((16,), e, jnp.int32)
                    wexp = plsc.load_gather(w_v, [erows, col0])
                    for q in range(4):
                        t_v[e, pl.ds(16 * q, 16)] = (
                            t_v[e, pl.ds(16 * q, 16)] * wexp)

        issue(0, 0)

        def blk_body(j, _):
            issue(1, 2 * j + 1)
            wait_gathers(0)
            compute_scatter(0)
            d0 = pltpu.async_copy(tb[0], acc.at[di[0]], semsc[0], add=True)
            wait_gathers(1)
            compute_scatter(1)
            d0.wait()

            @pl.when(j < half - 1)
            def _():
                issue(0, 2 * j + 2)

            d1 = pltpu.async_copy(tb[1], acc.at[di[1]], semsc[1], add=True)
            d1.wait()
            return 0

        lax.fori_loop(0, half, blk_body, 0)
        plsc.subcore_barrier()

        for k in range(rows_per_sub // EB):
            r0 = rowbase + k * EB
            pltpu.sync_copy(acc.at[pl.ds(r0, EB)], t0)
            pltpu.sync_copy(t0, out_hbm.at[c, pl.ds(r0, EB)])

    return sc_edge


_SC_EDGE_8 = _make_sc_edge(8)
_SC_EDGE_1 = _make_sc_edge(1)


def _edge_jnp(T, A, src, dst, g16, H):
    xp = T[:, :64]
    asrc = T[:, 64:64 + 8]
    adst = A[:, :8]
    t = asrc[src] + adst[dst]
    lr = jnp.maximum(t, 0.2 * t)
    w = jnp.exp(lr - g16[None, :8])
    if H == 8:
        wexp = jnp.repeat(w, 8, axis=1)
    else:
        wexp = jnp.repeat(w[:, :1], 64, axis=1)
        w = w.at[:, 1:].set(0.0)
    msg = jnp.concatenate([xp[src] * wexp, w], axis=1)
    acc = jax.ops.segment_sum(msg, dst, num_segments=NP)
    return jnp.stack([acc, jnp.zeros_like(acc)])


def _block_diag_heads(a):
    H, C = a.shape
    return (a[:, :, None] * jnp.eye(H, dtype=a.dtype)[:, None, :]).reshape(H * C, H)


@jax.jit
def kernel(x, edge_index, W1, a_src1, a_dst1, b1, W2, a_src2, a_dst2, b2,
           Wlin, blin):
    f32 = jnp.float32
    xpad = jnp.pad(x, ((0, NP - N), (0, 0)))

    As1 = _block_diag_heads(a_src1)
    Ad1 = _block_diag_heads(a_dst1)
    wcat1 = jnp.zeros((128, 128), f32)
    wcat1 = wcat1.at[:, :64].set(W1)
    wcat1 = wcat1.at[:, 64:72].set(W1 @ As1)
    wcat1 = wcat1.at[:, 72:80].set(W1 @ Ad1)

    ra, gm1 = _tcA(xpad, wcat1)
    T1 = ra[:, :72]
    A1 = ra[:, 72:80]
    g1 = gm1[0, 64:72] + gm1[0, 72:80]
    g16_1 = jnp.tile(g1, 2)

    loop = jnp.arange(N, dtype=jnp.int32)
    pad = jnp.full((EP - E_TOT,), N, jnp.int32)
    src = jnp.concatenate([edge_index[0].astype(jnp.int32), loop, pad])
    dst = jnp.concatenate([edge_index[1].astype(jnp.int32), loop, pad])

    P1 = _SC_EDGE_8(T1, A1, src, dst, g16_1)

    rep8 = jnp.repeat(jnp.eye(8, dtype=f32), 8, axis=1)
    wcat2 = jnp.zeros((64, 128), f32)
    wcat2 = wcat2.at[:, :64].set(W2)
    wcat2 = wcat2.at[:, 64:65].set(W2 @ a_src2.T)
    wcat2 = wcat2.at[:, 72:73].set(W2 @ a_dst2.T)

    rb, gm2 = _tcB(P1, b1.reshape(1, 64), rep8, wcat2)
    T2 = rb[:, :72]
    A2 = rb[:, 72:80]
    g2 = gm2[0, 64] + gm2[0, 72]
    g16_2 = jnp.full((16,), g2, f32)

    P2 = _SC_EDGE_1(T2, A2, src, dst, g16_2)

    pick8 = jnp.zeros((8, 64), f32).at[0, :].set(1.0)
    wlpad = jnp.zeros((64, 128), f32).at[:, :NOUT].set(Wlin)
    blpad = jnp.full((1, 128), NEG, f32).at[0, :NOUT].set(blin)

    rc = _tcC(P2, b2.reshape(1, 64), pick8, wlpad, blpad)
    return rc[:N, :NOUT]


# speedup vs baseline: 1.0205x; 1.0205x over previous
import functools
import jax
import jax.numpy as jnp
from jax import lax
from jax.experimental import pallas as pl
from jax.experimental.pallas import tpu as pltpu
from jax.experimental.pallas import tpu_sc as plsc

N = 10000
NP = 10240
D = 128
NOUT = 10
ROWB = 256
GRID = NP // ROWB

NEG = -1e30
E_TOT = 330000

NTILES = 32
EB = 128
EP = 335872
NBLK = EP // EB
WT = 72


def _tcA_body(x_ref, w_ref, t_ref, a_ref, gm_ref):
    i = pl.program_id(0)
    r = jnp.dot(x_ref[...], w_ref[...], preferred_element_type=jnp.float32)
    t_ref[...] = r[:, :WT]
    a_ref[...] = r[:, WT:WT + 8]
    cm = jnp.max(r, axis=0, keepdims=True)
    @pl.when(i == 0)
    def _():
        gm_ref[...] = jnp.full_like(gm_ref, NEG)
    gm_ref[0:1, :] = jnp.maximum(gm_ref[0:1, :], cm)


def _tcA(xpad, wcat):
    return pl.pallas_call(
        _tcA_body,
        grid=(GRID,),
        in_specs=[
            pl.BlockSpec((ROWB, 128), lambda i: (i, 0)),
            pl.BlockSpec((128, 128), lambda i: (0, 0)),
        ],
        out_specs=[
            pl.BlockSpec((ROWB, WT), lambda i: (i, 0)),
            pl.BlockSpec((ROWB, 8), lambda i: (i, 0)),
            pl.BlockSpec((8, 128), lambda i: (0, 0)),
        ],
        out_shape=[
            jax.ShapeDtypeStruct((NP, WT), jnp.float32),
            jax.ShapeDtypeStruct((NP, 8), jnp.float32),
            jax.ShapeDtypeStruct((8, 128), jnp.float32),
        ],
        compiler_params=pltpu.CompilerParams(
            dimension_semantics=("arbitrary",)),
    )(xpad, wcat)


def _tcB_body(p_ref, b_ref, rep_ref, w2_ref, t_ref, a_ref, gm_ref):
    i = pl.program_id(0)
    r1 = p_ref[0] + p_ref[1]
    acc = r1[:, :64]
    den = r1[:, 64:72]
    drep = jnp.dot(den, rep_ref[...], preferred_element_type=jnp.float32)
    h1 = jnp.maximum(acc / (drep + 1e-16) + b_ref[0:1, :], 0.0)
    r = jnp.dot(h1, w2_ref[...], preferred_element_type=jnp.float32)
    t_ref[...] = r[:, :WT]
    a_ref[...] = r[:, WT:WT + 8]
    cm = jnp.max(r, axis=0, keepdims=True)
    @pl.when(i == 0)
    def _():
        gm_ref[...] = jnp.full_like(gm_ref, NEG)
    gm_ref[0:1, :] = jnp.maximum(gm_ref[0:1, :], cm)


def _tcB(p, b1, rep8, w2cat):
    return pl.pallas_call(
        _tcB_body,
        grid=(GRID,),
        in_specs=[
            pl.BlockSpec((2, ROWB, 72), lambda i: (0, i, 0)),
            pl.BlockSpec((1, 64), lambda i: (0, 0)),
            pl.BlockSpec((8, 64), lambda i: (0, 0)),
            pl.BlockSpec((64, 128), lambda i: (0, 0)),
        ],
        out_specs=[
            pl.BlockSpec((ROWB, WT), lambda i: (i, 0)),
            pl.BlockSpec((ROWB, 8), lambda i: (i, 0)),
            pl.BlockSpec((8, 128), lambda i: (0, 0)),
        ],
        out_shape=[
            jax.ShapeDtypeStruct((NP, WT), jnp.float32),
            jax.ShapeDtypeStruct((NP, 8), jnp.float32),
            jax.ShapeDtypeStruct((8, 128), jnp.float32),
        ],
        compiler_params=pltpu.CompilerParams(
            dimension_semantics=("arbitrary",)),
    )(p, b1, rep8, w2cat)


def _tcC_body(p_ref, b_ref, pick_ref, wl_ref, bl_ref, out_ref):
    r2 = p_ref[0] + p_ref[1]
    acc = r2[:, :64]
    den8 = r2[:, 64:72]
    d = jnp.dot(den8, pick_ref[...], preferred_element_type=jnp.float32)
    out2 = jnp.maximum(acc / (d + 1e-16) + b_ref[0:1, :], 0.0)
    logits = jnp.dot(out2, wl_ref[...], preferred_element_type=jnp.float32)
    logits = logits + bl_ref[0:1, :]
    m = jnp.max(logits, axis=1, keepdims=True)
    s = logits - m
    lse = jnp.log(jnp.sum(jnp.exp(s), axis=1, keepdims=True))
    out_ref[...] = s - lse


def _tcC(p, b2, pick8, wlpad, blpad):
    return pl.pallas_call(
        _tcC_body,
        grid=(GRID,),
        in_specs=[
            pl.BlockSpec((2, ROWB, 72), lambda i: (0, i, 0)),
            pl.BlockSpec((1, 64), lambda i: (0, 0)),
            pl.BlockSpec((8, 64), lambda i: (0, 0)),
            pl.BlockSpec((64, 128), lambda i: (0, 0)),
            pl.BlockSpec((1, 128), lambda i: (0, 0)),
        ],
        out_specs=pl.BlockSpec((ROWB, 128), lambda i: (i, 0)),
        out_shape=jax.ShapeDtypeStruct((NP, 128), jnp.float32),
        compiler_params=pltpu.CompilerParams(
            dimension_semantics=("arbitrary",)),
    )(p, b2, pick8, wlpad, blpad)


def _make_sc_edge(H):
    mesh = plsc.VectorSubcoreMesh(core_axis_name="c", subcore_axis_name="s")
    nper = NBLK // NTILES
    half = nper // 2
    rows_per_sub = NP // 16

    buf_t = [pltpu.VMEM((EB, WT), jnp.float32)] * 2
    buf_a = [pltpu.VMEM((EB, 8), jnp.float32)] * 2
    buf_si = [pltpu.VMEM((EB,), jnp.int32)] * 2
    buf_di = [pltpu.VMEM((EB,), jnp.int32)] * 2

    @functools.partial(
        pl.kernel,
        out_type=jax.ShapeDtypeStruct((2, NP, WT), jnp.float32),
        mesh=mesh,
        scratch_types=buf_si + buf_di + buf_t + buf_a + [
            pltpu.VMEM((EB, 8), jnp.float32),
            pltpu.VMEM((16,), jnp.float32),
            pltpu.VMEM_SHARED((NP, WT), jnp.float32),
            pltpu.SemaphoreType.DMA,
            pltpu.SemaphoreType.DMA,
            pltpu.SemaphoreType.DMA,
            pltpu.SemaphoreType.DMA,
            pltpu.SemaphoreType.DMA,
            pltpu.SemaphoreType.DMA,
            pltpu.SemaphoreType.DMA,
        ],
        compiler_params=pltpu.CompilerParams(
            needs_layout_passes=False, use_tc_tiling_on_sc=False),
    )
    def sc_edge(t_hbm, a_hbm, src_hbm, dst_hbm, g_hbm, out_hbm,
                si0, si1, di0, di1, t0, t1, a0, a1,
                w_v, g_v, acc, semi, semt0, semt1, sema0, sema1,
                sems0, sems1):
        c = lax.axis_index("c")
        s = lax.axis_index("s")
        wid = s * 2 + c
        ii = lax.iota(jnp.int32, 16)
        zero16 = jnp.zeros((16,), jnp.float32)
        si = [si0, si1]
        di = [di0, di1]
        tb = [t0, t1]
        ab = [a0, a1]
        semt = [semt0, semt1]
        sema = [sema0, sema1]
        semsc = [sems0, sems1]

        pltpu.sync_copy(g_hbm, g_v)
        gvec = g_v[...]

        def _zrow(e, _):
            for q in range(4):
                t0[e, pl.ds(16 * q, 16)] = zero16
            return 0
        lax.fori_loop(0, EB, _zrow, 0)

        def _ztail(i, _):
            rows = 2 * i + jnp.where(ii >= 8, 1, 0)
            plsc.store_scatter(t0, [rows, 64 + (ii & 7)], zero16)
            return 0
        lax.fori_loop(0, EB // 2, _ztail, 0)

        rowbase = s * rows_per_sub
        for k in range(rows_per_sub // EB):
            pltpu.sync_copy(t0, acc.at[pl.ds(rowbase + k * EB, EB)])
        plsc.subcore_barrier()

        pair_rows = jnp.where(ii >= 8, 1, 0)
        head_cols = ii & 7
        col64 = jnp.full((16,), 64, jnp.int32)
        col0 = jnp.zeros((16,), jnp.int32)

        def issue(b, k):
            base = (k * NTILES + wid) * EB
            c1 = pltpu.async_copy(src_hbm.at[pl.ds(base, EB)], si[b], semi)
            c2 = pltpu.async_copy(dst_hbm.at[pl.ds(base, EB)], di[b], semi)
            c1.wait()
            c2.wait()
            pltpu.async_copy(t_hbm.at[si[b]], tb[b], semt[b])
            pltpu.async_copy(a_hbm.at[di[b]], ab[b], sema[b])

        def wait_gathers(b):
            pltpu.make_async_copy(t_hbm.at[si[b]], tb[b], semt[b]).wait()
            pltpu.make_async_copy(a_hbm.at[di[b]], ab[b], sema[b]).wait()

        def compute_scatter(b):
            t_v = tb[b]
            a_v = ab[b]
            if H == 8:
                @plsc.parallel_loop(0, EB // 2, unroll=4)
                def wbody(i):
                    rows = 2 * i + pair_rows
                    asv = plsc.load_gather(t_v, [rows, 64 + head_cols])
                    adv = plsc.load_gather(a_v, [rows, head_cols])
                    t = asv + adv
                    t = jnp.maximum(t, 0.2 * t)
                    w = jnp.exp(t - gvec)
                    plsc.store_scatter(w_v, [rows, head_cols], w)
                    plsc.store_scatter(t_v, [rows, 64 + head_cols], w)

                @plsc.parallel_loop(0, EB, unroll=8)
                def mbody(e):
                    erows = jnp.full((16,), e, jnp.int32)
                    for q in range(4):
                        wexp = plsc.load_gather(
                            w_v, [erows, 2 * q + pair_rows])
                        t_v[e, pl.ds(16 * q, 16)] = (
                            t_v[e, pl.ds(16 * q, 16)] * wexp)
            else:
                @plsc.parallel_loop(0, EB // 16, unroll=2)
                def wbody(i):
                    rows = 16 * i + ii
                    asv = plsc.load_gather(t_v, [rows, col64])
                    adv = plsc.load_gather(a_v, [rows, col0])
                    t = asv + adv
                    t = jnp.maximum(t, 0.2 * t)
                    w = jnp.exp(t - gvec)
                    plsc.store_scatter(w_v, [rows, col0], w)
                    plsc.store_scatter(t_v, [rows, col64], w)

                @plsc.parallel_loop(0, EB, unroll=8)
                def mbody(e):
                    erows = jnp.full((16,), e, jnp.int32)
                    wexp = plsc.load_gather(w_v, [erows, col0])
                    for q in range(4):
                        t_v[e, pl.ds(16 * q, 16)] = (
                            t_v[e, pl.ds(16 * q, 16)] * wexp)

        issue(0, 0)

        def blk_body(j, _):
            issue(1, 2 * j + 1)
            wait_gathers(0)
            compute_scatter(0)
            pltpu.sync_copy(tb[0], acc.at[di[0]], add=True)

            @pl.when(j < half - 1)
            def _():
                issue(0, 2 * j + 2)

            wait_gathers(1)
            compute_scatter(1)
            pltpu.sync_copy(tb[1], acc.at[di[1]], add=True)
            return 0

        lax.fori_loop(0, half, blk_body, 0)
        plsc.subcore_barrier()

        for k in range(rows_per_sub // EB):
            r0 = rowbase + k * EB
            pltpu.sync_copy(acc.at[pl.ds(r0, EB)], t0)
            pltpu.sync_copy(t0, out_hbm.at[c, pl.ds(r0, EB)])

    return sc_edge


_SC_EDGE_8 = _make_sc_edge(8)
_SC_EDGE_1 = _make_sc_edge(1)


def _edge_jnp(T, A, src, dst, g16, H):
    xp = T[:, :64]
    asrc = T[:, 64:64 + 8]
    adst = A[:, :8]
    t = asrc[src] + adst[dst]
    lr = jnp.maximum(t, 0.2 * t)
    w = jnp.exp(lr - g16[None, :8])
    if H == 8:
        wexp = jnp.repeat(w, 8, axis=1)
    else:
        wexp = jnp.repeat(w[:, :1], 64, axis=1)
        w = w.at[:, 1:].set(0.0)
    msg = jnp.concatenate([xp[src] * wexp, w], axis=1)
    acc = jax.ops.segment_sum(msg, dst, num_segments=NP)
    return jnp.stack([acc, jnp.zeros_like(acc)])


def _block_diag_heads(a):
    H, C = a.shape
    return (a[:, :, None] * jnp.eye(H, dtype=a.dtype)[:, None, :]).reshape(H * C, H)


@jax.jit
def kernel(x, edge_index, W1, a_src1, a_dst1, b1, W2, a_src2, a_dst2, b2,
           Wlin, blin):
    f32 = jnp.float32
    xpad = jnp.pad(x, ((0, NP - N), (0, 0)))

    As1 = _block_diag_heads(a_src1)
    Ad1 = _block_diag_heads(a_dst1)
    wcat1 = jnp.zeros((128, 128), f32)
    wcat1 = wcat1.at[:, :64].set(W1)
    wcat1 = wcat1.at[:, 64:72].set(W1 @ As1)
    wcat1 = wcat1.at[:, 72:80].set(W1 @ Ad1)

    T1, A1, gm1 = _tcA(xpad, wcat1)
    g1 = gm1[0, 64:72] + gm1[0, 72:80]
    g16_1 = jnp.tile(g1, 2)

    loop = jnp.arange(N, dtype=jnp.int32)
    pad = jnp.full((EP - E_TOT,), N, jnp.int32)
    src = jnp.concatenate([edge_index[0].astype(jnp.int32), loop, pad])
    dst = jnp.concatenate([edge_index[1].astype(jnp.int32), loop, pad])

    P1 = _SC_EDGE_8(T1, A1, src, dst, g16_1)

    rep8 = jnp.repeat(jnp.eye(8, dtype=f32), 8, axis=1)
    wcat2 = jnp.zeros((64, 128), f32)
    wcat2 = wcat2.at[:, :64].set(W2)
    wcat2 = wcat2.at[:, 64:65].set(W2 @ a_src2.T)
    wcat2 = wcat2.at[:, 72:73].set(W2 @ a_dst2.T)

    T2, A2, gm2 = _tcB(P1, b1.reshape(1, 64), rep8, wcat2)
    g2 = gm2[0, 64] + gm2[0, 72]
    g16_2 = jnp.full((16,), g2, f32)

    P2 = _SC_EDGE_1(T2, A2, src, dst, g16_2)

    pick8 = jnp.zeros((8, 64), f32).at[0, :].set(1.0)
    wlpad = jnp.zeros((64, 128), f32).at[:, :NOUT].set(Wlin)
    blpad = jnp.full((1, 128), NEG, f32).at[0, :NOUT].set(blin)

    rc = _tcC(P2, b2.reshape(1, 64), pick8, wlpad, blpad)
    return rc[:N, :NOUT]


# speedup vs baseline: 1.0216x; 1.0011x over previous
import functools
import jax
import jax.numpy as jnp
from jax import lax
from jax.experimental import pallas as pl
from jax.experimental.pallas import tpu as pltpu
from jax.experimental.pallas import tpu_sc as plsc

N = 10000
NP = 10240
D = 128
NOUT = 10
ROWB = 256
GRID = NP // ROWB

NEG = -1e30
E_TOT = 330000

NTILES = 32
EB = 128
EP = 335872
NBLK = EP // EB
WT = 72


def _tcA_body(x_ref, w_ref, t_ref, a_ref, gm_ref):
    i = pl.program_id(0)
    r = jnp.dot(x_ref[...], w_ref[...], preferred_element_type=jnp.float32)
    t_ref[...] = r[:, :WT]
    a_ref[...] = r[:, WT:WT + 8]
    cm = jnp.max(r, axis=0, keepdims=True)
    @pl.when(i == 0)
    def _():
        gm_ref[...] = jnp.full_like(gm_ref, NEG)
    gm_ref[0:1, :] = jnp.maximum(gm_ref[0:1, :], cm)


def _tcA(xpad, wcat):
    return pl.pallas_call(
        _tcA_body,
        grid=(GRID,),
        in_specs=[
            pl.BlockSpec((ROWB, 128), lambda i: (i, 0)),
            pl.BlockSpec((128, 128), lambda i: (0, 0)),
        ],
        out_specs=[
            pl.BlockSpec((ROWB, WT), lambda i: (i, 0)),
            pl.BlockSpec((ROWB, 8), lambda i: (i, 0)),
            pl.BlockSpec((8, 128), lambda i: (0, 0)),
        ],
        out_shape=[
            jax.ShapeDtypeStruct((NP, WT), jnp.float32),
            jax.ShapeDtypeStruct((NP, 8), jnp.float32),
            jax.ShapeDtypeStruct((8, 128), jnp.float32),
        ],
        compiler_params=pltpu.CompilerParams(
            dimension_semantics=("arbitrary",)),
    )(xpad, wcat)


def _tcB_body(p_ref, b_ref, rep_ref, w2_ref, t_ref, a_ref, gm_ref):
    i = pl.program_id(0)
    r1 = p_ref[0] + p_ref[1]
    acc = r1[:, :64]
    den = r1[:, 64:72]
    drep = jnp.dot(den, rep_ref[...], preferred_element_type=jnp.float32)
    h1 = jnp.maximum(acc / (drep + 1e-16) + b_ref[0:1, :], 0.0)
    r = jnp.dot(h1, w2_ref[...], preferred_element_type=jnp.float32)
    t_ref[...] = r[:, :WT]
    a_ref[...] = r[:, WT:WT + 8]
    cm = jnp.max(r, axis=0, keepdims=True)
    @pl.when(i == 0)
    def _():
        gm_ref[...] = jnp.full_like(gm_ref, NEG)
    gm_ref[0:1, :] = jnp.maximum(gm_ref[0:1, :], cm)


def _tcB(p, b1, rep8, w2cat):
    return pl.pallas_call(
        _tcB_body,
        grid=(GRID,),
        in_specs=[
            pl.BlockSpec((2, ROWB, 72), lambda i: (0, i, 0)),
            pl.BlockSpec((1, 64), lambda i: (0, 0)),
            pl.BlockSpec((8, 64), lambda i: (0, 0)),
            pl.BlockSpec((64, 128), lambda i: (0, 0)),
        ],
        out_specs=[
            pl.BlockSpec((ROWB, WT), lambda i: (i, 0)),
            pl.BlockSpec((ROWB, 8), lambda i: (i, 0)),
            pl.BlockSpec((8, 128), lambda i: (0, 0)),
        ],
        out_shape=[
            jax.ShapeDtypeStruct((NP, WT), jnp.float32),
            jax.ShapeDtypeStruct((NP, 8), jnp.float32),
            jax.ShapeDtypeStruct((8, 128), jnp.float32),
        ],
        compiler_params=pltpu.CompilerParams(
            dimension_semantics=("arbitrary",)),
    )(p, b1, rep8, w2cat)


def _tcC_body(p_ref, b_ref, pick_ref, wl_ref, bl_ref, out_ref):
    r2 = p_ref[0] + p_ref[1]
    acc = r2[:, :64]
    den8 = r2[:, 64:72]
    d = jnp.dot(den8, pick_ref[...], preferred_element_type=jnp.float32)
    out2 = jnp.maximum(acc / (d + 1e-16) + b_ref[0:1, :], 0.0)
    logits = jnp.dot(out2, wl_ref[...], preferred_element_type=jnp.float32)
    logits = logits + bl_ref[0:1, :]
    m = jnp.max(logits, axis=1, keepdims=True)
    s = logits - m
    lse = jnp.log(jnp.sum(jnp.exp(s), axis=1, keepdims=True))
    out_ref[...] = s - lse


def _tcC(p, b2, pick8, wlpad, blpad):
    return pl.pallas_call(
        _tcC_body,
        grid=(GRID,),
        in_specs=[
            pl.BlockSpec((2, ROWB, 72), lambda i: (0, i, 0)),
            pl.BlockSpec((1, 64), lambda i: (0, 0)),
            pl.BlockSpec((8, 64), lambda i: (0, 0)),
            pl.BlockSpec((64, 128), lambda i: (0, 0)),
            pl.BlockSpec((1, 128), lambda i: (0, 0)),
        ],
        out_specs=pl.BlockSpec((ROWB, 128), lambda i: (i, 0)),
        out_shape=jax.ShapeDtypeStruct((NP, 128), jnp.float32),
        compiler_params=pltpu.CompilerParams(
            dimension_semantics=("arbitrary",)),
    )(p, b2, pick8, wlpad, blpad)


def _make_sc_edge(H):
    mesh = plsc.VectorSubcoreMesh(core_axis_name="c", subcore_axis_name="s")
    nper = NBLK // NTILES
    half = nper // 2
    rows_per_sub = NP // 16

    buf_t = [pltpu.VMEM((EB, WT), jnp.float32)] * 2
    buf_a = [pltpu.VMEM((EB, 8), jnp.float32)] * 2
    buf_si = [pltpu.VMEM((EB,), jnp.int32)] * 2
    buf_di = [pltpu.VMEM((EB,), jnp.int32)] * 2

    @functools.partial(
        pl.kernel,
        out_type=jax.ShapeDtypeStruct((2, NP, WT), jnp.float32),
        mesh=mesh,
        scratch_types=buf_si + buf_di + buf_t + buf_a + [
            pltpu.VMEM((EB, 8), jnp.float32),
            pltpu.VMEM((16,), jnp.float32),
            pltpu.VMEM_SHARED((NP, WT), jnp.float32),
            pltpu.SemaphoreType.DMA,
            pltpu.SemaphoreType.DMA,
            pltpu.SemaphoreType.DMA,
            pltpu.SemaphoreType.DMA,
            pltpu.SemaphoreType.DMA,
            pltpu.SemaphoreType.DMA,
            pltpu.SemaphoreType.DMA,
        ],
        compiler_params=pltpu.CompilerParams(
            needs_layout_passes=False, use_tc_tiling_on_sc=False),
    )
    def sc_edge(t_hbm, a_hbm, src_hbm, dst_hbm, g_hbm, out_hbm,
                si0, si1, di0, di1, t0, t1, a0, a1,
                w_v, g_v, acc, semi, semt0, semt1, sema0, sema1,
                sems0, sems1):
        c = lax.axis_index("c")
        s = lax.axis_index("s")
        wid = s * 2 + c
        ii = lax.iota(jnp.int32, 16)
        zero16 = jnp.zeros((16,), jnp.float32)
        si = [si0, si1]
        di = [di0, di1]
        tb = [t0, t1]
        ab = [a0, a1]
        semt = [semt0, semt1]
        sema = [sema0, sema1]
        semsc = [sems0, sems1]

        pltpu.sync_copy(g_hbm, g_v)
        gvec = g_v[...]

        def _zrow(e, _):
            for q in range(4):
                t0[e, pl.ds(16 * q, 16)] = zero16
            return 0
        lax.fori_loop(0, EB, _zrow, 0)

        def _ztail(i, _):
            rows = 2 * i + jnp.where(ii >= 8, 1, 0)
            plsc.store_scatter(t0, [rows, 64 + (ii & 7)], zero16)
            return 0
        lax.fori_loop(0, EB // 2, _ztail, 0)

        rowbase = s * rows_per_sub
        for k in range(rows_per_sub // EB):
            pltpu.sync_copy(t0, acc.at[pl.ds(rowbase + k * EB, EB)])
        plsc.subcore_barrier()

        pair_rows = jnp.where(ii >= 8, 1, 0)
        head_cols = ii & 7
        col64 = jnp.full((16,), 64, jnp.int32)
        col0 = jnp.zeros((16,), jnp.int32)

        def issue(b, k):
            base = (k * NTILES + wid) * EB
            c1 = pltpu.async_copy(src_hbm.at[pl.ds(base, EB)], si[b], semi)
            c2 = pltpu.async_copy(dst_hbm.at[pl.ds(base, EB)], di[b], semi)
            c1.wait()
            c2.wait()
            pltpu.async_copy(t_hbm.at[si[b]], tb[b], semt[b])
            pltpu.async_copy(a_hbm.at[di[b]], ab[b], sema[b])

        def wait_gathers(b):
            pltpu.make_async_copy(t_hbm.at[si[b]], tb[b], semt[b]).wait()
            pltpu.make_async_copy(a_hbm.at[di[b]], ab[b], sema[b]).wait()

        def compute_scatter(b):
            t_v = tb[b]
            a_v = ab[b]
            if H == 8:
                @plsc.parallel_loop(0, EB // 2, unroll=4)
                def wbody(i):
                    rows = 2 * i + pair_rows
                    asv = plsc.load_gather(t_v, [rows, 64 + head_cols])
                    adv = plsc.load_gather(a_v, [rows, head_cols])
                    t = asv + adv
                    t = jnp.maximum(t, 0.2 * t)
                    w = jnp.exp(t - gvec)
                    plsc.store_scatter(w_v, [rows, head_cols], w)
                    plsc.store_scatter(t_v, [rows, 64 + head_cols], w)

                @plsc.parallel_loop(0, EB, unroll=8)
                def mbody(e):
                    erows = jnp.full((16,), e, jnp.int32)
                    for q in range(4):
                        wexp = plsc.load_gather(
                            w_v, [erows, 2 * q + pair_rows])
                        t_v[e, pl.ds(16 * q, 16)] = (
                            t_v[e, pl.ds(16 * q, 16)] * wexp)
            else:
                @plsc.parallel_loop(0, EB // 16, unroll=2)
                def wbody(i):
                    rows = 16 * i + ii
                    asv = plsc.load_gather(t_v, [rows, col64])
                    adv = plsc.load_gather(a_v, [rows, col0])
                    t = asv + adv
                    t = jnp.maximum(t, 0.2 * t)
                    w = jnp.exp(t - gvec)
                    plsc.store_scatter(w_v, [rows, col0], w)
                    plsc.store_scatter(t_v, [rows, col64], w)

                @plsc.parallel_loop(0, EB, unroll=8)
                def mbody(e):
                    erows = jnp.full((16,), e, jnp.int32)
                    wexp = plsc.load_gather(w_v, [erows, col0])
                    for q in range(4):
                        t_v[e, pl.ds(16 * q, 16)] = (
                            t_v[e, pl.ds(16 * q, 16)] * wexp)

        issue(0, 0)

        def blk_body(j, _):
            issue(1, 2 * j + 1)
            wait_gathers(0)
            compute_scatter(0)
            pltpu.sync_copy(tb[0], acc.at[di[0]], add=True)

            @pl.when(j < half - 1)
            def _():
                issue(0, 2 * j + 2)

            wait_gathers(1)
            compute_scatter(1)
            pltpu.sync_copy(tb[1], acc.at[di[1]], add=True)
            return 0

        lax.fori_loop(0, half, blk_body, 0)
        plsc.subcore_barrier()

        pltpu.sync_copy(acc.at[pl.ds(rowbase, rows_per_sub)],
                        out_hbm.at[c, pl.ds(rowbase, rows_per_sub)])

    return sc_edge


_SC_EDGE_8 = _make_sc_edge(8)
_SC_EDGE_1 = _make_sc_edge(1)


def _edge_jnp(T, A, src, dst, g16, H):
    xp = T[:, :64]
    asrc = T[:, 64:64 + 8]
    adst = A[:, :8]
    t = asrc[src] + adst[dst]
    lr = jnp.maximum(t, 0.2 * t)
    w = jnp.exp(lr - g16[None, :8])
    if H == 8:
        wexp = jnp.repeat(w, 8, axis=1)
    else:
        wexp = jnp.repeat(w[:, :1], 64, axis=1)
        w = w.at[:, 1:].set(0.0)
    msg = jnp.concatenate([xp[src] * wexp, w], axis=1)
    acc = jax.ops.segment_sum(msg, dst, num_segments=NP)
    return jnp.stack([acc, jnp.zeros_like(acc)])


def _block_diag_heads(a):
    H, C = a.shape
    return (a[:, :, None] * jnp.eye(H, dtype=a.dtype)[:, None, :]).reshape(H * C, H)


@jax.jit
def kernel(x, edge_index, W1, a_src1, a_dst1, b1, W2, a_src2, a_dst2, b2,
           Wlin, blin):
    f32 = jnp.float32
    xpad = jnp.pad(x, ((0, NP - N), (0, 0)))

    As1 = _block_diag_heads(a_src1)
    Ad1 = _block_diag_heads(a_dst1)
    wcat1 = jnp.zeros((128, 128), f32)
    wcat1 = wcat1.at[:, :64].set(W1)
    wcat1 = wcat1.at[:, 64:72].set(W1 @ As1)
    wcat1 = wcat1.at[:, 72:80].set(W1 @ Ad1)

    T1, A1, gm1 = _tcA(xpad, wcat1)
    g1 = gm1[0, 64:72] + gm1[0, 72:80]
    g16_1 = jnp.tile(g1, 2)

    loop = jnp.arange(N, dtype=jnp.int32)
    pad = jnp.full((EP - E_TOT,), N, jnp.int32)
    src = jnp.concatenate([edge_index[0].astype(jnp.int32), loop, pad])
    dst = jnp.concatenate([edge_index[1].astype(jnp.int32), loop, pad])

    P1 = _SC_EDGE_8(T1, A1, src, dst, g16_1)

    rep8 = jnp.repeat(jnp.eye(8, dtype=f32), 8, axis=1)
    wcat2 = jnp.zeros((64, 128), f32)
    wcat2 = wcat2.at[:, :64].set(W2)
    wcat2 = wcat2.at[:, 64:65].set(W2 @ a_src2.T)
    wcat2 = wcat2.at[:, 72:73].set(W2 @ a_dst2.T)

    T2, A2, gm2 = _tcB(P1, b1.reshape(1, 64), rep8, wcat2)
    g2 = gm2[0, 64] + gm2[0, 72]
    g16_2 = jnp.full((16,), g2, f32)

    P2 = _SC_EDGE_1(T2, A2, src, dst, g16_2)

    pick8 = jnp.zeros((8, 64), f32).at[0, :].set(1.0)
    wlpad = jnp.zeros((64, 128), f32).at[:, :NOUT].set(Wlin)
    blpad = jnp.full((1, 128), NEG, f32).at[0, :NOUT].set(blin)

    rc = _tcC(P2, b2.reshape(1, 64), pick8, wlpad, blpad)
    return rc[:N, :NOUT]


# speedup vs baseline: 1.0223x; 1.0006x over previous
import functools
import jax
import jax.numpy as jnp
from jax import lax
from jax.experimental import pallas as pl
from jax.experimental.pallas import tpu as pltpu
from jax.experimental.pallas import tpu_sc as plsc

N = 10000
NP = 10240
D = 128
NOUT = 10
ROWB = 256
GRID = NP // ROWB

NEG = -1e30
E_TOT = 330000

NTILES = 32
EB = 128
EP = 335872
NBLK = EP // EB
WT = 72


def _tcA_body(x_ref, w_ref, t_ref, a_ref, gm_ref):
    i = pl.program_id(0)
    r = jnp.dot(x_ref[...], w_ref[...], preferred_element_type=jnp.float32)
    t_ref[...] = r[:, :WT]
    a_ref[...] = r[:, WT:WT + 8]
    cm = jnp.max(r, axis=0, keepdims=True)
    @pl.when(i == 0)
    def _():
        gm_ref[...] = jnp.full_like(gm_ref, NEG)
    gm_ref[0:1, :] = jnp.maximum(gm_ref[0:1, :], cm)


def _tcA(xpad, wcat):
    return pl.pallas_call(
        _tcA_body,
        grid=(GRID,),
        in_specs=[
            pl.BlockSpec((ROWB, 128), lambda i: (i, 0)),
            pl.BlockSpec((128, 128), lambda i: (0, 0)),
        ],
        out_specs=[
            pl.BlockSpec((ROWB, WT), lambda i: (i, 0)),
            pl.BlockSpec((ROWB, 8), lambda i: (i, 0)),
            pl.BlockSpec((8, 128), lambda i: (0, 0)),
        ],
        out_shape=[
            jax.ShapeDtypeStruct((NP, WT), jnp.float32),
            jax.ShapeDtypeStruct((NP, 8), jnp.float32),
            jax.ShapeDtypeStruct((8, 128), jnp.float32),
        ],
        compiler_params=pltpu.CompilerParams(
            dimension_semantics=("arbitrary",)),
    )(xpad, wcat)


def _tcB_body(p_ref, b_ref, rep_ref, w2_ref, t_ref, a_ref, gm_ref):
    i = pl.program_id(0)
    r1 = p_ref[0] + p_ref[1]
    acc = r1[:, :64]
    den = r1[:, 64:72]
    drep = jnp.dot(den, rep_ref[...], preferred_element_type=jnp.float32)
    h1 = jnp.maximum(acc / (drep + 1e-16) + b_ref[0:1, :], 0.0)
    r = jnp.dot(h1, w2_ref[...], preferred_element_type=jnp.float32)
    t_ref[...] = r[:, :WT]
    a_ref[...] = r[:, WT:WT + 8]
    cm = jnp.max(r, axis=0, keepdims=True)
    @pl.when(i == 0)
    def _():
        gm_ref[...] = jnp.full_like(gm_ref, NEG)
    gm_ref[0:1, :] = jnp.maximum(gm_ref[0:1, :], cm)


def _tcB(p, b1, rep8, w2cat):
    return pl.pallas_call(
        _tcB_body,
        grid=(GRID,),
        in_specs=[
            pl.BlockSpec((2, ROWB, 72), lambda i: (0, i, 0)),
            pl.BlockSpec((1, 64), lambda i: (0, 0)),
            pl.BlockSpec((8, 64), lambda i: (0, 0)),
            pl.BlockSpec((64, 128), lambda i: (0, 0)),
        ],
        out_specs=[
            pl.BlockSpec((ROWB, WT), lambda i: (i, 0)),
            pl.BlockSpec((ROWB, 8), lambda i: (i, 0)),
            pl.BlockSpec((8, 128), lambda i: (0, 0)),
        ],
        out_shape=[
            jax.ShapeDtypeStruct((NP, WT), jnp.float32),
            jax.ShapeDtypeStruct((NP, 8), jnp.float32),
            jax.ShapeDtypeStruct((8, 128), jnp.float32),
        ],
        compiler_params=pltpu.CompilerParams(
            dimension_semantics=("arbitrary",)),
    )(p, b1, rep8, w2cat)


def _tcC_body(p_ref, b_ref, pick_ref, wl_ref, bl_ref, out_ref):
    r2 = p_ref[0] + p_ref[1]
    acc = r2[:, :64]
    den8 = r2[:, 64:72]
    d = jnp.dot(den8, pick_ref[...], preferred_element_type=jnp.float32)
    out2 = jnp.maximum(acc / (d + 1e-16) + b_ref[0:1, :], 0.0)
    logits = jnp.dot(out2, wl_ref[...], preferred_element_type=jnp.float32)
    logits = logits + bl_ref[0:1, :]
    m = jnp.max(logits, axis=1, keepdims=True)
    s = logits - m
    lse = jnp.log(jnp.sum(jnp.exp(s), axis=1, keepdims=True))
    out_ref[...] = s - lse


def _tcC(p, b2, pick8, wlpad, blpad):
    return pl.pallas_call(
        _tcC_body,
        grid=(GRID,),
        in_specs=[
            pl.BlockSpec((2, ROWB, 72), lambda i: (0, i, 0)),
            pl.BlockSpec((1, 64), lambda i: (0, 0)),
            pl.BlockSpec((8, 64), lambda i: (0, 0)),
            pl.BlockSpec((64, 128), lambda i: (0, 0)),
            pl.BlockSpec((1, 128), lambda i: (0, 0)),
        ],
        out_specs=pl.BlockSpec((ROWB, 128), lambda i: (i, 0)),
        out_shape=jax.ShapeDtypeStruct((NP, 128), jnp.float32),
        compiler_params=pltpu.CompilerParams(
            dimension_semantics=("arbitrary",)),
    )(p, b2, pick8, wlpad, blpad)


def _make_sc_edge(H):
    mesh = plsc.VectorSubcoreMesh(core_axis_name="c", subcore_axis_name="s")
    nper = NBLK // NTILES
    half = nper // 2
    rows_per_sub = NP // 16

    buf_t = [pltpu.VMEM((EB, WT), jnp.float32)] * 2
    buf_a = [pltpu.VMEM((EB, 8), jnp.float32)] * 2
    buf_si = [pltpu.VMEM((EB,), jnp.int32)] * 2
    buf_di = [pltpu.VMEM((EB,), jnp.int32)] * 2

    @functools.partial(
        pl.kernel,
        out_type=jax.ShapeDtypeStruct((2, NP, WT), jnp.float32),
        mesh=mesh,
        scratch_types=buf_si + buf_di + buf_t + buf_a + [
            pltpu.VMEM((EB, 8), jnp.float32),
            pltpu.VMEM((16,), jnp.float32),
            pltpu.VMEM_SHARED((NP, WT), jnp.float32),
            pltpu.SemaphoreType.DMA,
            pltpu.SemaphoreType.DMA,
            pltpu.SemaphoreType.DMA,
            pltpu.SemaphoreType.DMA,
            pltpu.SemaphoreType.DMA,
            pltpu.SemaphoreType.DMA,
            pltpu.SemaphoreType.DMA,
        ],
        compiler_params=pltpu.CompilerParams(
            needs_layout_passes=False, use_tc_tiling_on_sc=False),
    )
    def sc_edge(t_hbm, a_hbm, src_hbm, dst_hbm, g_hbm, out_hbm,
                si0, si1, di0, di1, t0, t1, a0, a1,
                w_v, g_v, acc, semi, semt0, semt1, sema0, sema1,
                sems0, sems1):
        c = lax.axis_index("c")
        s = lax.axis_index("s")
        wid = s * 2 + c
        ii = lax.iota(jnp.int32, 16)
        zero16 = jnp.zeros((16,), jnp.float32)
        si = [si0, si1]
        di = [di0, di1]
        tb = [t0, t1]
        ab = [a0, a1]
        semt = [semt0, semt1]
        sema = [sema0, sema1]
        semsc = [sems0, sems1]

        pltpu.sync_copy(g_hbm, g_v)
        gvec = g_v[...]

        @plsc.parallel_loop(0, EB, unroll=4)
        def _zrow(e):
            for q in range(4):
                t0[e, pl.ds(16 * q, 16)] = zero16

        @plsc.parallel_loop(0, EB // 2, unroll=4)
        def _ztail(i):
            rows = 2 * i + jnp.where(ii >= 8, 1, 0)
            plsc.store_scatter(t0, [rows, 64 + (ii & 7)], zero16)

        rowbase = s * rows_per_sub
        for k in range(rows_per_sub // EB):
            pltpu.sync_copy(t0, acc.at[pl.ds(rowbase + k * EB, EB)])
        plsc.subcore_barrier()

        pair_rows = jnp.where(ii >= 8, 1, 0)
        head_cols = ii & 7
        col64 = jnp.full((16,), 64, jnp.int32)
        col0 = jnp.zeros((16,), jnp.int32)

        def issue(b, k):
            base = (k * NTILES + wid) * EB
            c1 = pltpu.async_copy(src_hbm.at[pl.ds(base, EB)], si[b], semi)
            c2 = pltpu.async_copy(dst_hbm.at[pl.ds(base, EB)], di[b], semi)
            c1.wait()
            c2.wait()
            pltpu.async_copy(t_hbm.at[si[b]], tb[b], semt[b])
            pltpu.async_copy(a_hbm.at[di[b]], ab[b], sema[b])

        def wait_gathers(b):
            pltpu.make_async_copy(t_hbm.at[si[b]], tb[b], semt[b]).wait()
            pltpu.make_async_copy(a_hbm.at[di[b]], ab[b], sema[b]).wait()

        def compute_scatter(b):
            t_v = tb[b]
            a_v = ab[b]
            if H == 8:
                @plsc.parallel_loop(0, EB // 2, unroll=4)
                def wbody(i):
                    rows = 2 * i + pair_rows
                    asv = plsc.load_gather(t_v, [rows, 64 + head_cols])
                    adv = plsc.load_gather(a_v, [rows, head_cols])
                    t = asv + adv
                    t = jnp.maximum(t, 0.2 * t)
                    w = jnp.exp(t - gvec)
                    plsc.store_scatter(w_v, [rows, head_cols], w)
                    plsc.store_scatter(t_v, [rows, 64 + head_cols], w)

                @plsc.parallel_loop(0, EB, unroll=8)
                def mbody(e):
                    erows = jnp.full((16,), e, jnp.int32)
                    for q in range(4):
                        wexp = plsc.load_gather(
                            w_v, [erows, 2 * q + pair_rows])
                        t_v[e, pl.ds(16 * q, 16)] = (
                            t_v[e, pl.ds(16 * q, 16)] * wexp)
            else:
                @plsc.parallel_loop(0, EB // 16, unroll=2)
                def wbody(i):
                    rows = 16 * i + ii
                    asv = plsc.load_gather(t_v, [rows, col64])
                    adv = plsc.load_gather(a_v, [rows, col0])
                    t = asv + adv
                    t = jnp.maximum(t, 0.2 * t)
                    w = jnp.exp(t - gvec)
                    plsc.store_scatter(w_v, [rows, col0], w)
                    plsc.store_scatter(t_v, [rows, col64], w)

                @plsc.parallel_loop(0, EB, unroll=8)
                def mbody(e):
                    erows = jnp.full((16,), e, jnp.int32)
                    wexp = plsc.load_gather(w_v, [erows, col0])
                    for q in range(4):
                        t_v[e, pl.ds(16 * q, 16)] = (
                            t_v[e, pl.ds(16 * q, 16)] * wexp)

        issue(0, 0)

        def blk_body(j, _):
            issue(1, 2 * j + 1)
            wait_gathers(0)
            compute_scatter(0)
            pltpu.sync_copy(tb[0], acc.at[di[0]], add=True)

            @pl.when(j < half - 1)
            def _():
                issue(0, 2 * j + 2)

            wait_gathers(1)
            compute_scatter(1)
            pltpu.sync_copy(tb[1], acc.at[di[1]], add=True)
            return 0

        lax.fori_loop(0, half, blk_body, 0)
        plsc.subcore_barrier()

        pltpu.sync_copy(acc.at[pl.ds(rowbase, rows_per_sub)],
                        out_hbm.at[c, pl.ds(rowbase, rows_per_sub)])

    return sc_edge


_SC_EDGE_8 = _make_sc_edge(8)
_SC_EDGE_1 = _make_sc_edge(1)


def _block_diag_heads(a):
    H, C = a.shape
    return (a[:, :, None] * jnp.eye(H, dtype=a.dtype)[:, None, :]).reshape(H * C, H)


@jax.jit
def kernel(x, edge_index, W1, a_src1, a_dst1, b1, W2, a_src2, a_dst2, b2,
           Wlin, blin):
    f32 = jnp.float32
    xpad = jnp.pad(x, ((0, NP - N), (0, 0)))

    As1 = _block_diag_heads(a_src1)
    Ad1 = _block_diag_heads(a_dst1)
    wcat1 = jnp.zeros((128, 128), f32)
    wcat1 = wcat1.at[:, :64].set(W1)
    wcat1 = wcat1.at[:, 64:72].set(W1 @ As1)
    wcat1 = wcat1.at[:, 72:80].set(W1 @ Ad1)

    T1, A1, gm1 = _tcA(xpad, wcat1)
    g1 = gm1[0, 64:72] + gm1[0, 72:80]
    g16_1 = jnp.tile(g1, 2)

    loop = jnp.arange(N, dtype=jnp.int32)
    pad = jnp.full((EP - E_TOT,), N, jnp.int32)
    src = jnp.concatenate([edge_index[0].astype(jnp.int32), loop, pad])
    dst = jnp.concatenate([edge_index[1].astype(jnp.int32), loop, pad])

    P1 = _SC_EDGE_8(T1, A1, src, dst, g16_1)

    rep8 = jnp.repeat(jnp.eye(8, dtype=f32), 8, axis=1)
    wcat2 = jnp.zeros((64, 128), f32)
    wcat2 = wcat2.at[:, :64].set(W2)
    wcat2 = wcat2.at[:, 64:65].set(W2 @ a_src2.T)
    wcat2 = wcat2.at[:, 72:73].set(W2 @ a_dst2.T)

    T2, A2, gm2 = _tcB(P1, b1.reshape(1, 64), rep8, wcat2)
    g2 = gm2[0, 64] + gm2[0, 72]
    g16_2 = jnp.full((16,), g2, f32)

    P2 = _SC_EDGE_1(T2, A2, src, dst, g16_2)

    pick8 = jnp.zeros((8, 64), f32).at[0, :].set(1.0)
    wlpad = jnp.zeros((64, 128), f32).at[:, :NOUT].set(Wlin)
    blpad = jnp.full((1, 128), NEG, f32).at[0, :NOUT].set(blin)

    rc = _tcC(P2, b2.reshape(1, 64), pick8, wlpad, blpad)
    return rc[:N, :NOUT]


# speedup vs baseline: 1.1170x; 1.0927x over previous
import functools
import jax
import jax.numpy as jnp
from jax import lax
from jax.experimental import pallas as pl
from jax.experimental.pallas import tpu as pltpu
from jax.experimental.pallas import tpu_sc as plsc

N = 10000
NP = 10240
D = 128
NOUT = 10
ROWB = 256
GRID = NP // ROWB

NEG = -1e30
E_TOT = 330000

NTILES = 32
EB = 128
EP = 335872
NBLK = EP // EB
WT = 72


def _tcA_body(x_ref, w_ref, t_ref, a_ref, gm_ref):
    i = pl.program_id(0)
    r = jnp.dot(x_ref[...], w_ref[...], preferred_element_type=jnp.float32)
    t_ref[...] = r[:, :WT]
    a_ref[...] = r[:, WT:WT + 8]
    cm = jnp.max(r, axis=0, keepdims=True)
    @pl.when(i == 0)
    def _():
        gm_ref[...] = jnp.full_like(gm_ref, NEG)
    gm_ref[0:1, :] = jnp.maximum(gm_ref[0:1, :], cm)


def _tcA(xpad, wcat):
    return pl.pallas_call(
        _tcA_body,
        grid=(GRID,),
        in_specs=[
            pl.BlockSpec((ROWB, 128), lambda i: (i, 0)),
            pl.BlockSpec((128, 128), lambda i: (0, 0)),
        ],
        out_specs=[
            pl.BlockSpec((ROWB, WT), lambda i: (i, 0)),
            pl.BlockSpec((ROWB, 8), lambda i: (i, 0)),
            pl.BlockSpec((8, 128), lambda i: (0, 0)),
        ],
        out_shape=[
            jax.ShapeDtypeStruct((NP, WT), jnp.float32),
            jax.ShapeDtypeStruct((NP, 8), jnp.float32),
            jax.ShapeDtypeStruct((8, 128), jnp.float32),
        ],
        compiler_params=pltpu.CompilerParams(
            dimension_semantics=("arbitrary",)),
    )(xpad, wcat)


def _tcB_body(p_ref, b_ref, rep_ref, w2_ref, t_ref, a_ref, gm_ref):
    i = pl.program_id(0)
    r1 = p_ref[0] + p_ref[1]
    acc = r1[:, :64]
    den = r1[:, 64:72]
    drep = jnp.dot(den, rep_ref[...], preferred_element_type=jnp.float32)
    h1 = jnp.maximum(acc / (drep + 1e-16) + b_ref[0:1, :], 0.0)
    r = jnp.dot(h1, w2_ref[...], preferred_element_type=jnp.float32)
    t_ref[...] = r[:, :WT]
    a_ref[...] = r[:, WT:WT + 8]
    cm = jnp.max(r, axis=0, keepdims=True)
    @pl.when(i == 0)
    def _():
        gm_ref[...] = jnp.full_like(gm_ref, NEG)
    gm_ref[0:1, :] = jnp.maximum(gm_ref[0:1, :], cm)


def _tcB(p, b1, rep8, w2cat):
    return pl.pallas_call(
        _tcB_body,
        grid=(GRID,),
        in_specs=[
            pl.BlockSpec((2, ROWB, 72), lambda i: (0, i, 0)),
            pl.BlockSpec((1, 64), lambda i: (0, 0)),
            pl.BlockSpec((8, 64), lambda i: (0, 0)),
            pl.BlockSpec((64, 128), lambda i: (0, 0)),
        ],
        out_specs=[
            pl.BlockSpec((ROWB, WT), lambda i: (i, 0)),
            pl.BlockSpec((ROWB, 8), lambda i: (i, 0)),
            pl.BlockSpec((8, 128), lambda i: (0, 0)),
        ],
        out_shape=[
            jax.ShapeDtypeStruct((NP, WT), jnp.float32),
            jax.ShapeDtypeStruct((NP, 8), jnp.float32),
            jax.ShapeDtypeStruct((8, 128), jnp.float32),
        ],
        compiler_params=pltpu.CompilerParams(
            dimension_semantics=("arbitrary",)),
    )(p, b1, rep8, w2cat)


def _tcC_body(p_ref, b_ref, pick_ref, wl_ref, bl_ref, out_ref):
    r2 = p_ref[0] + p_ref[1]
    acc = r2[:, :64]
    den8 = r2[:, 64:72]
    d = jnp.dot(den8, pick_ref[...], preferred_element_type=jnp.float32)
    out2 = jnp.maximum(acc / (d + 1e-16) + b_ref[0:1, :], 0.0)
    logits = jnp.dot(out2, wl_ref[...], preferred_element_type=jnp.float32)
    logits = logits + bl_ref[0:1, :]
    m = jnp.max(logits, axis=1, keepdims=True)
    s = logits - m
    lse = jnp.log(jnp.sum(jnp.exp(s), axis=1, keepdims=True))
    out_ref[...] = s - lse


def _tcC(p, b2, pick8, wlpad, blpad):
    return pl.pallas_call(
        _tcC_body,
        grid=(GRID,),
        in_specs=[
            pl.BlockSpec((2, ROWB, 72), lambda i: (0, i, 0)),
            pl.BlockSpec((1, 64), lambda i: (0, 0)),
            pl.BlockSpec((8, 64), lambda i: (0, 0)),
            pl.BlockSpec((64, 128), lambda i: (0, 0)),
            pl.BlockSpec((1, 128), lambda i: (0, 0)),
        ],
        out_specs=pl.BlockSpec((ROWB, 128), lambda i: (i, 0)),
        out_shape=jax.ShapeDtypeStruct((NP, 128), jnp.float32),
        compiler_params=pltpu.CompilerParams(
            dimension_semantics=("arbitrary",)),
    )(p, b2, pick8, wlpad, blpad)


def _make_sc_edge(H):
    mesh = plsc.VectorSubcoreMesh(core_axis_name="c", subcore_axis_name="s")
    nper = NBLK // NTILES
    half = nper // 2
    rows_per_sub = NP // 16

    buf_t = [pltpu.VMEM((EB, WT), jnp.float32)] * 2
    buf_a = [pltpu.VMEM((EB, 8), jnp.float32)] * 2

    @functools.partial(
        pl.kernel,
        out_type=jax.ShapeDtypeStruct((2, NP, WT), jnp.float32),
        mesh=mesh,
        scratch_types=buf_t + buf_a + [
            pltpu.VMEM((nper, EB), jnp.int32),
            pltpu.VMEM((nper, EB), jnp.int32),
            pltpu.VMEM((EB, 8), jnp.float32),
            pltpu.VMEM((16,), jnp.float32),
            pltpu.VMEM_SHARED((NP, WT), jnp.float32),
            pltpu.SemaphoreType.DMA,
            pltpu.SemaphoreType.DMA,
            pltpu.SemaphoreType.DMA,
            pltpu.SemaphoreType.DMA,
            pltpu.SemaphoreType.DMA,
        ],
        compiler_params=pltpu.CompilerParams(
            needs_layout_passes=False, use_tc_tiling_on_sc=False),
    )
    def sc_edge(t_hbm, a_hbm, src_hbm, dst_hbm, g_hbm, out_hbm,
                t0, t1, a0, a1, si_all, di_all,
                w_v, g_v, acc, semi, semt0, semt1, sema0, sema1):
        c = lax.axis_index("c")
        s = lax.axis_index("s")
        wid = s * 2 + c
        ii = lax.iota(jnp.int32, 16)
        zero16 = jnp.zeros((16,), jnp.float32)
        tb = [t0, t1]
        ab = [a0, a1]
        semt = [semt0, semt1]
        sema = [sema0, sema1]

        pltpu.sync_copy(g_hbm, g_v)
        gvec = g_v[...]

        c1 = pltpu.async_copy(src_hbm.at[wid], si_all, semi)
        c2 = pltpu.async_copy(dst_hbm.at[wid], di_all, semi)
        c1.wait()
        c2.wait()

        @plsc.parallel_loop(0, EB, unroll=4)
        def _zrow(e):
            for q in range(4):
                t0[e, pl.ds(16 * q, 16)] = zero16

        @plsc.parallel_loop(0, EB // 2, unroll=4)
        def _ztail(i):
            rows = 2 * i + jnp.where(ii >= 8, 1, 0)
            plsc.store_scatter(t0, [rows, 64 + (ii & 7)], zero16)

        rowbase = s * rows_per_sub
        for k in range(rows_per_sub // EB):
            pltpu.sync_copy(t0, acc.at[pl.ds(rowbase + k * EB, EB)])
        plsc.subcore_barrier()

        pair_rows = jnp.where(ii >= 8, 1, 0)
        head_cols = ii & 7
        col64 = jnp.full((16,), 64, jnp.int32)
        col0 = jnp.zeros((16,), jnp.int32)

        def issue(b, k):
            pltpu.async_copy(t_hbm.at[si_all.at[k]], tb[b], semt[b])
            pltpu.async_copy(a_hbm.at[di_all.at[k]], ab[b], sema[b])

        def wait_gathers(b, k):
            pltpu.make_async_copy(t_hbm.at[si_all.at[k]], tb[b], semt[b]).wait()
            pltpu.make_async_copy(a_hbm.at[di_all.at[k]], ab[b], sema[b]).wait()

        def compute_scatter(b):
            t_v = tb[b]
            a_v = ab[b]
            if H == 8:
                @plsc.parallel_loop(0, EB // 2, unroll=4)
                def wbody(i):
                    rows = 2 * i + pair_rows
                    asv = plsc.load_gather(t_v, [rows, 64 + head_cols])
                    adv = plsc.load_gather(a_v, [rows, head_cols])
                    t = asv + adv
                    t = jnp.maximum(t, 0.2 * t)
                    w = jnp.exp(t - gvec)
                    plsc.store_scatter(w_v, [rows, head_cols], w)
                    plsc.store_scatter(t_v, [rows, 64 + head_cols], w)

                @plsc.parallel_loop(0, EB, unroll=8)
                def mbody(e):
                    erows = jnp.full((16,), e, jnp.int32)
                    for q in range(4):
                        wexp = plsc.load_gather(
                            w_v, [erows, 2 * q + pair_rows])
                        t_v[e, pl.ds(16 * q, 16)] = (
                            t_v[e, pl.ds(16 * q, 16)] * wexp)
            else:
                @plsc.parallel_loop(0, EB // 16, unroll=2)
                def wbody(i):
                    rows = 16 * i + ii
                    asv = plsc.load_gather(t_v, [rows, col64])
                    adv = plsc.load_gather(a_v, [rows, col0])
                    t = asv + adv
                    t = jnp.maximum(t, 0.2 * t)
                    w = jnp.exp(t - gvec)
                    plsc.store_scatter(w_v, [rows, col0], w)
                    plsc.store_scatter(t_v, [rows, col64], w)

                @plsc.parallel_loop(0, EB, unroll=8)
                def mbody(e):
                    erows = jnp.full((16,), e, jnp.int32)
                    wexp = plsc.load_gather(w_v, [erows, col0])
                    for q in range(4):
                        t_v[e, pl.ds(16 * q, 16)] = (
                            t_v[e, pl.ds(16 * q, 16)] * wexp)

        issue(0, 0)

        def blk_body(j, _):
            issue(1, 2 * j + 1)
            wait_gathers(0, 2 * j)
            compute_scatter(0)
            pltpu.sync_copy(tb[0], acc.at[di_all.at[2 * j]], add=True)

            @pl.when(j < half - 1)
            def _():
                issue(0, 2 * j + 2)

            wait_gathers(1, 2 * j + 1)
            compute_scatter(1)
            pltpu.sync_copy(tb[1], acc.at[di_all.at[2 * j + 1]], add=True)
            return 0

        lax.fori_loop(0, half, blk_body, 0)
        plsc.subcore_barrier()

        pltpu.sync_copy(acc.at[pl.ds(rowbase, rows_per_sub)],
                        out_hbm.at[c, pl.ds(rowbase, rows_per_sub)])

    return sc_edge


_SC_EDGE_8 = _make_sc_edge(8)
_SC_EDGE_1 = _make_sc_edge(1)


def _block_diag_heads(a):
    H, C = a.shape
    return (a[:, :, None] * jnp.eye(H, dtype=a.dtype)[:, None, :]).reshape(H * C, H)


@jax.jit
def kernel(x, edge_index, W1, a_src1, a_dst1, b1, W2, a_src2, a_dst2, b2,
           Wlin, blin):
    f32 = jnp.float32
    xpad = jnp.pad(x, ((0, NP - N), (0, 0)))

    As1 = _block_diag_heads(a_src1)
    Ad1 = _block_diag_heads(a_dst1)
    wcat1 = jnp.zeros((128, 128), f32)
    wcat1 = wcat1.at[:, :64].set(W1)
    wcat1 = wcat1.at[:, 64:72].set(W1 @ As1)
    wcat1 = wcat1.at[:, 72:80].set(W1 @ Ad1)

    T1, A1, gm1 = _tcA(xpad, wcat1)
    g1 = gm1[0, 64:72] + gm1[0, 72:80]
    g16_1 = jnp.tile(g1, 2)

    loop = jnp.arange(N, dtype=jnp.int32)
    pad = jnp.full((EP - E_TOT,), N, jnp.int32)
    src = jnp.concatenate([edge_index[0].astype(jnp.int32), loop, pad])
    dst = jnp.concatenate([edge_index[1].astype(jnp.int32), loop, pad])
    nper = EP // (NTILES * EB)
    src = src.reshape(nper, NTILES, EB).transpose(1, 0, 2)
    dst = dst.reshape(nper, NTILES, EB).transpose(1, 0, 2)

    P1 = _SC_EDGE_8(T1, A1, src, dst, g16_1)

    rep8 = jnp.repeat(jnp.eye(8, dtype=f32), 8, axis=1)
    wcat2 = jnp.zeros((64, 128), f32)
    wcat2 = wcat2.at[:, :64].set(W2)
    wcat2 = wcat2.at[:, 64:65].set(W2 @ a_src2.T)
    wcat2 = wcat2.at[:, 72:73].set(W2 @ a_dst2.T)

    T2, A2, gm2 = _tcB(P1, b1.reshape(1, 64), rep8, wcat2)
    g2 = gm2[0, 64] + gm2[0, 72]
    g16_2 = jnp.full((16,), g2, f32)

    P2 = _SC_EDGE_1(T2, A2, src, dst, g16_2)

    pick8 = jnp.zeros((8, 64), f32).at[0, :].set(1.0)
    wlpad = jnp.zeros((64, 128), f32).at[:, :NOUT].set(Wlin)
    blpad = jnp.full((1, 128), NEG, f32).at[0, :NOUT].set(blin)

    rc = _tcC(P2, b2.reshape(1, 64), pick8, wlpad, blpad)
    return rc[:N, :NOUT]
